# Initial kernel scaffold; baseline (speedup 1.0000x reference)
import jax
import jax.numpy as jnp
from jax.experimental import pallas as pl


def kernel(gat_w, gat_att_src, gat_att_dst, gat_b, gcn1_w, gcn1_b, gcn3_w, gcn3_b, bn1_gamma, bn1_beta, bn1_mean, bn1_var, bn2_gamma, bn2_beta, bn2_mean, bn2_var, lk0_w, lk0_b, lk1_w, lk1_b, lk2_w, lk2_b, x, edge_index, edge_attr, edge_index_test):
    raise NotImplementedError("write your pallas kernel here")


# speedup vs baseline: 2.3521x; 2.3521x over previous
import functools

import jax
import jax.numpy as jnp
from jax import lax
from jax.experimental import pallas as pl
from jax.experimental.pallas import tpu as pltpu

_VMEM = 64 * 1024 * 1024 - 16 * 1024 * 1024


def _ceil_to(n, m):
    return ((n + m - 1) // m) * m


def _pad2(a, shape):
    return jnp.pad(a, [(0, t - s) for s, t in zip(a.shape, shape)])


def _gat_fused_kernel(x_ref, xt_ref, wg_ref, am_ref, asrcT_ref, mask_ref,
                      s1_ref, h1_ref, w1_ref, o_ref, *, heads, fh):
    xw_all = jnp.dot(x_ref[...], wg_ref[...],
                     preferred_element_type=jnp.float32)
    e_srcT = lax.dot_general(asrcT_ref[...], xw_all,
                             (((1,), (1,)), ((), ())),
                             preferred_element_type=jnp.float32)
    tq = o_ref.shape[0]
    xw_tile = jnp.dot(xt_ref[...], wg_ref[...],
                      preferred_element_type=jnp.float32)
    e_dst = jnp.dot(xw_tile, am_ref[...],
                    preferred_element_type=jnp.float32)
    mask = mask_ref[...].astype(jnp.float32)

    cols = []
    for h in range(heads):
        s = e_dst[:, h:h + 1] + e_srcT[h:h + 1, :]
        s = jnp.maximum(s, 0.2 * s) + mask
        m = jnp.max(s, axis=-1, keepdims=True)
        p = jnp.exp(s - m)
        l = jnp.sum(p, axis=-1, keepdims=True)
        num = jnp.dot(p.astype(jnp.bfloat16),
                      xw_all[:, h * fh:(h + 1) * fh].astype(jnp.bfloat16),
                      preferred_element_type=jnp.float32)
        cols.append(num / l)
    pad = o_ref.shape[1] - heads * fh
    if pad:
        cols.append(jnp.zeros((tq, pad), jnp.float32))
    y = jnp.concatenate(cols, axis=-1)
    y = jnp.maximum(y * s1_ref[...] + h1_ref[...], 0.0)
    o_ref[...] = jnp.dot(y, w1_ref[...],
                         preferred_element_type=jnp.float32)


def _run_gat(x, wg, am, asrcT, mask, s1, h1, w1, *, heads, fh, tq):
    N = x.shape[0]
    K = x.shape[1]
    HFP = wg.shape[1]
    kern = functools.partial(_gat_fused_kernel, heads=heads, fh=fh)
    return pl.pallas_call(
        kern,
        grid=(N // tq,),
        in_specs=[
            pl.BlockSpec((N, K), lambda i: (0, 0)),
            pl.BlockSpec((tq, K), lambda i: (i, 0)),
            pl.BlockSpec((K, HFP), lambda i: (0, 0)),
            pl.BlockSpec((HFP, 128), lambda i: (0, 0)),
            pl.BlockSpec((8, HFP), lambda i: (0, 0)),
            pl.BlockSpec((tq, N), lambda i: (i, 0)),
            pl.BlockSpec((1, HFP), lambda i: (0, 0)),
            pl.BlockSpec((1, HFP), lambda i: (0, 0)),
            pl.BlockSpec((HFP, HFP), lambda i: (0, 0)),
        ],
        out_specs=pl.BlockSpec((tq, HFP), lambda i: (i, 0)),
        out_shape=jax.ShapeDtypeStruct((N, HFP), jnp.float32),
        compiler_params=pltpu.CompilerParams(
            dimension_semantics=("parallel",),
            vmem_limit_bytes=_VMEM),
    )(x, x, wg, am, asrcT, mask, s1, h1, w1)


def _gcn_kernel(a_ref, xw_ref, xwd_ref, dv_ref, dvd_ref, b_ref, s2_ref,
                h2_ref, wn_ref, o_ref, acc_ref, *, channels, fh,
                relu_before_affine, project_out):
    k = pl.program_id(1)

    @pl.when(k == 0)
    def _():
        acc_ref[...] = jnp.zeros(acc_ref.shape, jnp.float32)

    xw = (xw_ref[...] * dv_ref[...]).astype(jnp.bfloat16)
    for c in range(channels):
        lo, hi = c * fh, (c + 1) * fh
        acc_ref[:, lo:hi] += jnp.dot(
            a_ref[c], xw[:, lo:hi], preferred_element_type=jnp.float32)

    @pl.when(k == pl.num_programs(1) - 1)
    def _():
        dvd = dvd_ref[...]
        agg = dvd * acc_ref[...] + dvd * dvd * xwd_ref[...] + b_ref[...]
        if relu_before_affine:
            y = jnp.maximum(agg, 0.0) * s2_ref[...] + h2_ref[...]
        else:
            y = jnp.maximum(agg * s2_ref[...] + h2_ref[...], 0.0)
        if project_out:
            y = jnp.dot(y, wn_ref[...], preferred_element_type=jnp.float32)
        o_ref[...] = y


def _run_gcn(A, xw, dv, b, s2, h2, wn, *, channels, fh, relu_before_affine,
             project_out, tm, tk):
    C, N, _ = A.shape
    CFP = xw.shape[1]
    kern = functools.partial(_gcn_kernel, channels=channels, fh=fh,
                             relu_before_affine=relu_before_affine,
                             project_out=project_out)
    return pl.pallas_call(
        kern,
        grid=(N // tm, N // tk),
        in_specs=[
            pl.BlockSpec((C, tm, tk), lambda i, k: (0, i, k)),
            pl.BlockSpec((tk, CFP), lambda i, k: (k, 0)),
            pl.BlockSpec((tm, CFP), lambda i, k: (i, 0)),
            pl.BlockSpec((tk, CFP), lambda i, k: (k, 0)),
            pl.BlockSpec((tm, CFP), lambda i, k: (i, 0)),
            pl.BlockSpec((1, CFP), lambda i, k: (0, 0)),
            pl.BlockSpec((1, CFP), lambda i, k: (0, 0)),
            pl.BlockSpec((1, CFP), lambda i, k: (0, 0)),
            pl.BlockSpec((CFP, CFP), lambda i, k: (0, 0)),
        ],
        out_specs=pl.BlockSpec((tm, CFP), lambda i, k: (i, 0)),
        out_shape=jax.ShapeDtypeStruct((N, CFP), jnp.float32),
        scratch_shapes=[pltpu.VMEM((tm, CFP), jnp.float32)],
        compiler_params=pltpu.CompilerParams(
            dimension_semantics=("parallel", "arbitrary"),
            vmem_limit_bytes=_VMEM),
    )(A, xw, xw, dv, dv, b, s2, h2, wn)


def _mlp_kernel(ef_ref, eb_ref, w1_ref, b1_ref, w2_ref, b2_ref, w3_ref,
                b3_ref, o_ref):
    def _net(e):
        h = jnp.maximum(jnp.dot(e, w1_ref[...],
                                preferred_element_type=jnp.float32)
                        + b1_ref[...], 0.0)
        h = jnp.maximum(jnp.dot(h, w2_ref[...],
                                preferred_element_type=jnp.float32)
                        + b2_ref[...], 0.0)
        return jnp.dot(h, w3_ref[...],
                       preferred_element_type=jnp.float32) + b3_ref[...]

    o_ref[...] = 0.5 * (_net(ef_ref[...]) + _net(eb_ref[...]))


def _run_mlp(both, w1, b1, w2, b2, w3, b3, *, te):
    R2, F = both.shape
    R = R2 // 2
    HLP = w1.shape[1]
    OUTP = w3.shape[1]
    nb = R // te
    return pl.pallas_call(
        _mlp_kernel,
        grid=(nb,),
        in_specs=[
            pl.BlockSpec((te, F), lambda i: (i, 0)),
            pl.BlockSpec((te, F), lambda i, _nb=nb: (i + _nb, 0)),
            pl.BlockSpec((F, HLP), lambda i: (0, 0)),
            pl.BlockSpec((1, HLP), lambda i: (0, 0)),
            pl.BlockSpec((HLP, HLP), lambda i: (0, 0)),
            pl.BlockSpec((1, HLP), lambda i: (0, 0)),
            pl.BlockSpec((HLP, OUTP), lambda i: (0, 0)),
            pl.BlockSpec((1, OUTP), lambda i: (0, 0)),
        ],
        out_specs=pl.BlockSpec((te, OUTP), lambda i: (i, 0)),
        out_shape=jax.ShapeDtypeStruct((R, OUTP), jnp.float32),
        compiler_params=pltpu.CompilerParams(
            dimension_semantics=("parallel",),
            vmem_limit_bytes=_VMEM),
    )(both, both, w1, b1, w2, b2, w3, b3)


def kernel(gat_w, gat_att_src, gat_att_dst, gat_b, gcn1_w, gcn1_b, gcn3_w,
           gcn3_b, bn1_gamma, bn1_beta, bn1_mean, bn1_var, bn2_gamma,
           bn2_beta, bn2_mean, bn2_var, lk0_w, lk0_b, lk1_w, lk1_b, lk2_w,
           lk2_b, x, edge_index, edge_attr, edge_index_test):
    eps = 1e-5
    N = x.shape[0]
    C = edge_attr.shape[1]
    heads, fh = gat_att_src.shape[0], gat_att_src.shape[2]
    D1 = heads * fh
    D2 = C * fh
    HFP = _ceil_to(D1, 128)
    CFP = _ceil_to(D2, 128)

    flat = edge_index[1].astype(jnp.int32) * N + edge_index[0].astype(jnp.int32)
    w6 = jnp.concatenate(
        [edge_attr, jnp.ones((edge_attr.shape[0], 1), jnp.float32)], axis=1)
    planes = jnp.zeros((N * N, C + 1), jnp.float32).at[flat].add(w6)
    planes = planes.reshape(N, N, C + 1)
    Araw = planes[:, :, :C].transpose(2, 0, 1).astype(jnp.bfloat16)
    cnt = planes[:, :, C]

    deg = jnp.sum(planes[:, :, :C], axis=1) + 1.0
    dinv = lax.rsqrt(deg)
    dv = jnp.repeat(dinv, fh, axis=1)
    dv = _pad2(dv, (N, CFP))

    eye = jnp.eye(N, dtype=jnp.float32)
    mask = jnp.where(cnt + eye > 0.0, 0.0, -1e30).astype(jnp.bfloat16)

    scale1 = bn1_gamma * lax.rsqrt(bn1_var + eps)
    shift1 = (gat_b - bn1_mean) * scale1 + bn1_beta
    scale2 = bn2_gamma * lax.rsqrt(bn2_var + eps)
    shift2 = bn2_beta - bn2_mean * scale2
    s1 = _pad2(scale1.reshape(1, D1), (1, HFP))
    h1 = _pad2(shift1.reshape(1, D1), (1, HFP))
    s2 = _pad2(scale2.reshape(1, D2), (1, CFP))
    h2 = _pad2(shift2.reshape(1, D2), (1, CFP))

    wg = _pad2(gat_w, (x.shape[1], HFP))
    asrc = gat_att_src.reshape(heads, fh)
    adst = gat_att_dst.reshape(heads, fh)
    eyeh = jnp.eye(heads, dtype=jnp.float32)
    asrc_be = (eyeh[:, :, None] * asrc[:, None, :]).reshape(heads, D1)
    adst_be = (eyeh[:, :, None] * adst[:, None, :]).reshape(heads, D1)
    asrcT = _pad2(asrc_be, (8, HFP))
    am = _pad2(adst_be.T, (HFP, 128))

    w1_all = _pad2(gcn1_w.transpose(1, 0, 2).reshape(D1, D2), (HFP, CFP))
    b1_all = _pad2(gcn1_b.reshape(1, D2), (1, CFP))
    w3_all = _pad2(gcn3_w.transpose(1, 0, 2).reshape(D2, D2), (CFP, CFP))
    b3_all = _pad2(gcn3_b.reshape(1, D2), (1, CFP))

    tq = min(256, N)
    tm = min(256, N)
    tk = min(512, N)
    xw1 = _run_gat(x, wg, am, asrcT, mask, s1, h1, w1_all,
                   heads=heads, fh=fh, tq=tq)
    xw3 = _run_gcn(Araw, xw1, dv, b1_all, s2, h2, w3_all,
                   channels=C, fh=fh, relu_before_affine=False,
                   project_out=True, tm=tm, tk=tk)
    x3 = _run_gcn(Araw, xw3, dv, b3_all, s2, h2, w3_all,
                  channels=C, fh=fh, relu_before_affine=True,
                  project_out=False, tm=tm, tk=tk)

    src, dst = edge_index_test[0], edge_index_test[1]
    e_fwd = jnp.concatenate([x3[src], x3[dst]], axis=-1)
    e_bwd = jnp.concatenate([x3[dst], x3[src]], axis=-1)
    both = jnp.concatenate([e_fwd, e_bwd], axis=0)

    HL = lk0_w.shape[1]
    OUT = lk2_w.shape[1]
    HLP = _ceil_to(HL, 128)
    OUTP = _ceil_to(OUT, 128)
    w1m = jnp.zeros((2 * CFP, HLP), jnp.float32)
    w1m = w1m.at[:D2, :HL].set(lk0_w[:D2])
    w1m = w1m.at[CFP:CFP + D2, :HL].set(lk0_w[D2:])
    b1m = _pad2(lk0_b.reshape(1, HL), (1, HLP))
    w2m = _pad2(lk1_w, (HLP, HLP))
    b2m = _pad2(lk1_b.reshape(1, HL), (1, HLP))
    w3m = _pad2(lk2_w, (HLP, OUTP))
    b3m = _pad2(lk2_b.reshape(1, OUT), (1, OUTP))

    out = _run_mlp(both, w1m, b1m, w2m, b2m, w3m, b3m,
                   te=min(256, src.shape[0]))
    return out[:, :OUT]
